# baseline (speedup 1.0000x reference)
import functools

import jax
import jax.numpy as jnp
from jax import lax
from jax.experimental import pallas as pl
from jax.experimental.pallas import tpu as pltpu
from jax.experimental.pallas import tpu_sc as plsc

X_LEN = 33554432
I32_MAX = 2147483647

NC = 2
NS = 16
L = 16
NW = NC * NS
S_SC = 18874368
PER_W = S_SC // NW
CHUNK = 16384
NCHUNK = PER_W // CHUNK
NBUF = 4
U = 8
K = CHUNK // (L * U)

S_TC = X_LEN - S_SC
ROWS = X_LEN // 128
ROW0 = S_SC // 128
BM = 2048
GRID = S_TC // 128 // BM
UT = 8
TK = BM // (8 * UT)


def _sc_body(x_hbm, vals_hbm, idxs_hbm,
             buf0, buf1, buf2, buf3, vrow, irow,
             sem0, sem1, sem2, sem3):
    wid = lax.axis_index("s") * NC + lax.axis_index("c")
    base = wid * PER_W
    lane = lax.broadcasted_iota(jnp.int32, (L,), 0)
    bufs = (buf0, buf1, buf2, buf3)
    sems = (sem0, sem1, sem2, sem3)

    for b in range(NBUF):
        pltpu.async_copy(
            x_hbm.at[pl.ds(base + b * CHUNK, CHUNK)], bufs[b], sems[b])

    def make_inner(buf, ck):
        def inner(j, ic):
            bas, bcodes = ic
            codev = jnp.full((L,), ck + j, jnp.int32)
            nbas = []
            ncodes = []
            for u in range(U):
                v = buf[pl.ds(j * (L * U) + u * L, L)]
                a = jnp.abs(v)
                pred = a < bas[u]
                nbas.append(jnp.minimum(a, bas[u]))
                ncodes.append(jnp.where(pred, codev, bcodes[u]))
            return tuple(nbas), tuple(ncodes)
        return inner

    ngroup = NCHUNK // NBUF

    def group_step(g, carry):
        bas, bcodes = carry
        for b in range(NBUF):
            c = NBUF * g + b
            pltpu.make_async_copy(
                x_hbm.at[pl.ds(base, CHUNK)], bufs[b], sems[b]).wait()
            bas, bcodes = lax.fori_loop(
                0, K, make_inner(bufs[b], c * K), (bas, bcodes))

            @pl.when(g < ngroup - 1)
            def _():
                off_next = base + (c + NBUF) * CHUNK
                pltpu.async_copy(
                    x_hbm.at[pl.ds(off_next, CHUNK)], bufs[b], sems[b])
        return bas, bcodes

    init_a = tuple(jnp.full((L,), jnp.inf, jnp.float32) for _ in range(U))
    init_c = tuple(jnp.zeros((L,), jnp.int32) for _ in range(U))
    bas, bcodes = lax.fori_loop(0, ngroup, group_step, (init_a, init_c))

    best_a = bas[0]
    best_p = base + bcodes[0] * (L * U) + lane
    for u in range(1, U):
        p_u = base + bcodes[u] * (L * U) + (u * L) + lane
        pred = (bas[u] < best_a) | ((bas[u] == best_a) & (p_u < best_p))
        best_a = jnp.where(pred, bas[u], best_a)
        best_p = jnp.where(pred, p_u, best_p)

    vrow[...] = best_a
    irow[...] = best_p
    pltpu.sync_copy(vrow, vals_hbm.at[wid])
    pltpu.sync_copy(irow, idxs_hbm.at[wid])


def _tc_body(x_ref, val_ref, idx_ref, *state):
    i = pl.program_id(0)
    sv = state[:UT]
    scd = state[UT:]

    @pl.when(i == 0)
    def _():
        for u in range(UT):
            sv[u][...] = jnp.full((8, 128), jnp.inf, jnp.float32)
            scd[u][...] = jnp.zeros((8, 128), jnp.int32)

    def inner(j, carry):
        bvs, bcs = carry
        codev = jnp.full((8, 128), i * TK + j, jnp.int32)
        nv, nc = [], []
        for u in range(UT):
            v = x_ref[pl.ds((j * UT + u) * 8, 8), :]
            a = jnp.abs(v)
            pred = a < bvs[u]
            nv.append(jnp.minimum(a, bvs[u]))
            nc.append(jnp.where(pred, codev, bcs[u]))
        return tuple(nv), tuple(nc)

    bvs, bcs = lax.fori_loop(
        0, TK, inner,
        (tuple(sv[u][...] for u in range(UT)),
         tuple(scd[u][...] for u in range(UT))))
    for u in range(UT):
        sv[u][...] = bvs[u]
        scd[u][...] = bcs[u]

    @pl.when(i == GRID - 1)
    def _():
        sub = lax.broadcasted_iota(jnp.int32, (8, 128), 0)
        col = lax.broadcasted_iota(jnp.int32, (8, 128), 1)
        best_v = bvs[0]
        best_p = S_SC + ((bcs[0] * UT + 0) * 8 + sub) * 128 + col
        for u in range(1, UT):
            p_u = S_SC + ((bcs[u] * UT + u) * 8 + sub) * 128 + col
            pred = (bvs[u] < best_v) | ((bvs[u] == best_v) & (p_u < best_p))
            best_v = jnp.where(pred, bvs[u], best_v)
            best_p = jnp.where(pred, p_u, best_p)
        vm = jnp.where((col & 1) == 0, best_v, jnp.inf)
        m = jnp.min(vm)
        sel = jnp.where(vm == m, best_p, I32_MAX)
        pbest = jnp.min(sel)
        val_ref[...] = jnp.reshape(m, (1, 1))
        idx_ref[...] = jnp.reshape(pbest, (1, 1))


def _merge_body(vals_ref, idxs_ref, tval_ref, tidx_ref, out_ref):
    v = vals_ref[...]
    p = idxs_ref[...]
    tv = tval_ref[0, 0]
    tp = tidx_ref[0, 0]
    col = lax.broadcasted_iota(jnp.int32, (NW, L), 1)
    even = (col & 1) == 0
    vm = jnp.where(even, v, jnp.inf)
    m = jnp.minimum(jnp.min(vm), tv)
    sel = jnp.where(vm == m, p, I32_MAX)
    p_sc = jnp.min(sel)
    p_tc = jnp.where(tv == m, tp, I32_MAX)
    p_best = jnp.minimum(p_sc, p_tc)
    out_ref[...] = jnp.reshape((p_best >> 1) + 1, (1, 1))


@jax.jit
def kernel(x):
    mesh = plsc.VectorSubcoreMesh(core_axis_name="c", subcore_axis_name="s")
    sc = functools.partial(
        pl.kernel,
        mesh=mesh,
        out_type=[
            jax.ShapeDtypeStruct((NW, L), jnp.float32),
            jax.ShapeDtypeStruct((NW, L), jnp.int32),
        ],
        scratch_types=[
            pltpu.VMEM((CHUNK,), jnp.float32),
            pltpu.VMEM((CHUNK,), jnp.float32),
            pltpu.VMEM((CHUNK,), jnp.float32),
            pltpu.VMEM((CHUNK,), jnp.float32),
            pltpu.VMEM((L,), jnp.float32),
            pltpu.VMEM((L,), jnp.int32),
            pltpu.SemaphoreType.DMA,
            pltpu.SemaphoreType.DMA,
            pltpu.SemaphoreType.DMA,
            pltpu.SemaphoreType.DMA,
        ],
    )(_sc_body)

    x2 = x.reshape(ROWS, 128)
    tval, tidx = pl.pallas_call(
        _tc_body,
        grid=(GRID,),
        in_specs=[pl.BlockSpec((BM, 128), lambda i: (ROW0 // BM + i, 0))],
        out_specs=[
            pl.BlockSpec((1, 1), lambda i: (0, 0)),
            pl.BlockSpec((1, 1), lambda i: (0, 0)),
        ],
        out_shape=[
            jax.ShapeDtypeStruct((1, 1), jnp.float32),
            jax.ShapeDtypeStruct((1, 1), jnp.int32),
        ],
        scratch_shapes=(
            [pltpu.VMEM((8, 128), jnp.float32) for _ in range(UT)]
            + [pltpu.VMEM((8, 128), jnp.int32) for _ in range(UT)]
        ),
    )(x2)
    vals, idxs = sc(x)

    out = pl.pallas_call(
        _merge_body,
        out_shape=jax.ShapeDtypeStruct((1, 1), jnp.int32),
    )(vals, idxs, tval, tidx)
    return out[0, 0]


# speedup vs baseline: 1.0834x; 1.0834x over previous
import functools

import jax
import jax.numpy as jnp
from jax import lax
from jax.experimental import pallas as pl
from jax.experimental.pallas import tpu as pltpu
from jax.experimental.pallas import tpu_sc as plsc

X_LEN = 33554432
I32_MAX = 2147483647

NC = 2
NS = 16
L = 16
NW = NC * NS
S_SC = 25165824
PER_W = S_SC // NW
CHUNK = 16384
NCHUNK = PER_W // CHUNK
NBUF = 4
U = 8
K = CHUNK // (L * U)

S_TC = X_LEN - S_SC
ROWS = X_LEN // 128
ROW0 = S_SC // 128
BM = 4096
GRID = S_TC // 128 // BM
UT = 8
TK = BM // (8 * UT)


def _sc_body(x_hbm, vals_hbm, idxs_hbm,
             buf0, buf1, buf2, buf3, vrow, irow,
             sem0, sem1, sem2, sem3):
    wid = lax.axis_index("s") * NC + lax.axis_index("c")
    base = wid * PER_W
    lane = lax.broadcasted_iota(jnp.int32, (L,), 0)
    bufs = (buf0, buf1, buf2, buf3)
    sems = (sem0, sem1, sem2, sem3)

    for b in range(NBUF):
        pltpu.async_copy(
            x_hbm.at[pl.ds(base + b * CHUNK, CHUNK)], bufs[b], sems[b])

    def make_inner(buf, ck):
        def inner(j, ic):
            bas, bcodes = ic
            codev = jnp.full((L,), ck + j, jnp.int32)
            nbas = []
            ncodes = []
            for u in range(U):
                v = buf[pl.ds(j * (L * U) + u * L, L)]
                a = jnp.abs(v)
                pred = a < bas[u]
                nbas.append(jnp.minimum(a, bas[u]))
                ncodes.append(jnp.where(pred, codev, bcodes[u]))
            return tuple(nbas), tuple(ncodes)
        return inner

    ngroup = NCHUNK // NBUF

    def group_step(g, carry):
        bas, bcodes = carry
        for b in range(NBUF):
            c = NBUF * g + b
            pltpu.make_async_copy(
                x_hbm.at[pl.ds(base, CHUNK)], bufs[b], sems[b]).wait()
            bas, bcodes = lax.fori_loop(
                0, K, make_inner(bufs[b], c * K), (bas, bcodes))

            @pl.when(g < ngroup - 1)
            def _():
                off_next = base + (c + NBUF) * CHUNK
                pltpu.async_copy(
                    x_hbm.at[pl.ds(off_next, CHUNK)], bufs[b], sems[b])
        return bas, bcodes

    init_a = tuple(jnp.full((L,), jnp.inf, jnp.float32) for _ in range(U))
    init_c = tuple(jnp.zeros((L,), jnp.int32) for _ in range(U))
    bas, bcodes = lax.fori_loop(0, ngroup, group_step, (init_a, init_c))

    best_a = bas[0]
    best_p = base + bcodes[0] * (L * U) + lane
    for u in range(1, U):
        p_u = base + bcodes[u] * (L * U) + (u * L) + lane
        pred = (bas[u] < best_a) | ((bas[u] == best_a) & (p_u < best_p))
        best_a = jnp.where(pred, bas[u], best_a)
        best_p = jnp.where(pred, p_u, best_p)

    vrow[...] = best_a
    irow[...] = best_p
    pltpu.sync_copy(vrow, vals_hbm.at[wid])
    pltpu.sync_copy(irow, idxs_hbm.at[wid])


def _tc_body(x_ref, val_ref, idx_ref, *state):
    i = pl.program_id(0)
    sv = state[:UT]
    scd = state[UT:]

    @pl.when(i == 0)
    def _():
        for u in range(UT):
            sv[u][...] = jnp.full((8, 128), jnp.inf, jnp.float32)
            scd[u][...] = jnp.zeros((8, 128), jnp.int32)

    def inner(j, carry):
        bvs, bcs = carry
        codev = jnp.full((8, 128), i * TK + j, jnp.int32)
        nv, nc = [], []
        for u in range(UT):
            v = x_ref[pl.ds((j * UT + u) * 8, 8), :]
            a = jnp.abs(v)
            pred = a < bvs[u]
            nv.append(jnp.minimum(a, bvs[u]))
            nc.append(jnp.where(pred, codev, bcs[u]))
        return tuple(nv), tuple(nc)

    bvs, bcs = lax.fori_loop(
        0, TK, inner,
        (tuple(sv[u][...] for u in range(UT)),
         tuple(scd[u][...] for u in range(UT))))
    for u in range(UT):
        sv[u][...] = bvs[u]
        scd[u][...] = bcs[u]

    @pl.when(i == GRID - 1)
    def _():
        sub = lax.broadcasted_iota(jnp.int32, (8, 128), 0)
        col = lax.broadcasted_iota(jnp.int32, (8, 128), 1)
        best_v = bvs[0]
        best_p = S_SC + ((bcs[0] * UT + 0) * 8 + sub) * 128 + col
        for u in range(1, UT):
            p_u = S_SC + ((bcs[u] * UT + u) * 8 + sub) * 128 + col
            pred = (bvs[u] < best_v) | ((bvs[u] == best_v) & (p_u < best_p))
            best_v = jnp.where(pred, bvs[u], best_v)
            best_p = jnp.where(pred, p_u, best_p)
        vm = jnp.where((col & 1) == 0, best_v, jnp.inf)
        m = jnp.min(vm)
        sel = jnp.where(vm == m, best_p, I32_MAX)
        pbest = jnp.min(sel)
        val_ref[...] = jnp.reshape(m, (1, 1))
        idx_ref[...] = jnp.reshape(pbest, (1, 1))


def _merge_body(vals_ref, idxs_ref, tval_ref, tidx_ref, out_ref):
    v = vals_ref[...]
    p = idxs_ref[...]
    tv = tval_ref[0, 0]
    tp = tidx_ref[0, 0]
    col = lax.broadcasted_iota(jnp.int32, (NW, L), 1)
    even = (col & 1) == 0
    vm = jnp.where(even, v, jnp.inf)
    m = jnp.minimum(jnp.min(vm), tv)
    sel = jnp.where(vm == m, p, I32_MAX)
    p_sc = jnp.min(sel)
    p_tc = jnp.where(tv == m, tp, I32_MAX)
    p_best = jnp.minimum(p_sc, p_tc)
    out_ref[...] = jnp.reshape((p_best >> 1) + 1, (1, 1))


@jax.jit
def kernel(x):
    mesh = plsc.VectorSubcoreMesh(core_axis_name="c", subcore_axis_name="s")
    sc = functools.partial(
        pl.kernel,
        mesh=mesh,
        out_type=[
            jax.ShapeDtypeStruct((NW, L), jnp.float32),
            jax.ShapeDtypeStruct((NW, L), jnp.int32),
        ],
        scratch_types=[
            pltpu.VMEM((CHUNK,), jnp.float32),
            pltpu.VMEM((CHUNK,), jnp.float32),
            pltpu.VMEM((CHUNK,), jnp.float32),
            pltpu.VMEM((CHUNK,), jnp.float32),
            pltpu.VMEM((L,), jnp.float32),
            pltpu.VMEM((L,), jnp.int32),
            pltpu.SemaphoreType.DMA,
            pltpu.SemaphoreType.DMA,
            pltpu.SemaphoreType.DMA,
            pltpu.SemaphoreType.DMA,
        ],
    )(_sc_body)

    x2 = x.reshape(ROWS, 128)
    tval, tidx = pl.pallas_call(
        _tc_body,
        grid=(GRID,),
        in_specs=[pl.BlockSpec((BM, 128), lambda i: (ROW0 // BM + i, 0))],
        out_specs=[
            pl.BlockSpec((1, 1), lambda i: (0, 0)),
            pl.BlockSpec((1, 1), lambda i: (0, 0)),
        ],
        out_shape=[
            jax.ShapeDtypeStruct((1, 1), jnp.float32),
            jax.ShapeDtypeStruct((1, 1), jnp.int32),
        ],
        scratch_shapes=(
            [pltpu.VMEM((8, 128), jnp.float32) for _ in range(UT)]
            + [pltpu.VMEM((8, 128), jnp.int32) for _ in range(UT)]
        ),
    )(x2)
    vals, idxs = sc(x)

    out = pl.pallas_call(
        _merge_body,
        out_shape=jax.ShapeDtypeStruct((1, 1), jnp.int32),
    )(vals, idxs, tval, tidx)
    return out[0, 0]


# speedup vs baseline: 1.1652x; 1.0755x over previous
import functools

import jax
import jax.numpy as jnp
from jax import lax
from jax.experimental import pallas as pl
from jax.experimental.pallas import tpu as pltpu
from jax.experimental.pallas import tpu_sc as plsc

X_LEN = 33554432
I32_MAX = 2147483647

NC = 2
NS = 16
L = 16
NW = NC * NS
S_SC = 18874368
PER_W = S_SC // NW
CHUNK = 16384
NCHUNK = PER_W // CHUNK
NBUF = 4
U = 8
K = CHUNK // (L * U)

S_TC = X_LEN - S_SC
ROWS = X_LEN // 128
ROW0 = S_SC // 128
BM = 4096
GRID = S_TC // 128 // BM
UT = 8
TK = BM // (8 * UT)


def _sc_body(x_hbm, vals_hbm, idxs_hbm,
             buf0, buf1, buf2, buf3, vrow, irow,
             sem0, sem1, sem2, sem3):
    wid = lax.axis_index("s") * NC + lax.axis_index("c")
    base = wid * PER_W
    lane = lax.broadcasted_iota(jnp.int32, (L,), 0)
    bufs = (buf0, buf1, buf2, buf3)
    sems = (sem0, sem1, sem2, sem3)

    for b in range(NBUF):
        pltpu.async_copy(
            x_hbm.at[pl.ds(base + b * CHUNK, CHUNK)], bufs[b], sems[b])

    def make_inner(buf, ck):
        def inner(j, ic):
            bas, bcodes = ic
            codev = jnp.full((L,), ck + j, jnp.int32)
            nbas = []
            ncodes = []
            for u in range(U):
                v = buf[pl.ds(j * (L * U) + u * L, L)]
                a = jnp.abs(v)
                pred = a < bas[u]
                nbas.append(jnp.minimum(a, bas[u]))
                ncodes.append(jnp.where(pred, codev, bcodes[u]))
            return tuple(nbas), tuple(ncodes)
        return inner

    ngroup = NCHUNK // NBUF

    def group_step(g, carry):
        bas, bcodes = carry
        for b in range(NBUF):
            c = NBUF * g + b
            pltpu.make_async_copy(
                x_hbm.at[pl.ds(base, CHUNK)], bufs[b], sems[b]).wait()
            bas, bcodes = lax.fori_loop(
                0, K, make_inner(bufs[b], c * K), (bas, bcodes))

            @pl.when(g < ngroup - 1)
            def _():
                off_next = base + (c + NBUF) * CHUNK
                pltpu.async_copy(
                    x_hbm.at[pl.ds(off_next, CHUNK)], bufs[b], sems[b])
        return bas, bcodes

    init_a = tuple(jnp.full((L,), jnp.inf, jnp.float32) for _ in range(U))
    init_c = tuple(jnp.zeros((L,), jnp.int32) for _ in range(U))
    bas, bcodes = lax.fori_loop(0, ngroup, group_step, (init_a, init_c))

    best_a = bas[0]
    best_p = base + bcodes[0] * (L * U) + lane
    for u in range(1, U):
        p_u = base + bcodes[u] * (L * U) + (u * L) + lane
        pred = (bas[u] < best_a) | ((bas[u] == best_a) & (p_u < best_p))
        best_a = jnp.where(pred, bas[u], best_a)
        best_p = jnp.where(pred, p_u, best_p)

    vrow[...] = best_a
    irow[...] = best_p
    pltpu.sync_copy(vrow, vals_hbm.at[wid])
    pltpu.sync_copy(irow, idxs_hbm.at[wid])


def _tc_body(x_ref, val_ref, idx_ref, *state):
    i = pl.program_id(0)
    sv = state[:UT]
    scd = state[UT:]

    @pl.when(i == 0)
    def _():
        for u in range(UT):
            sv[u][...] = jnp.full((8, 128), jnp.inf, jnp.float32)
            scd[u][...] = jnp.zeros((8, 128), jnp.int32)

    def inner(j, carry):
        bvs, bcs = carry
        codev = jnp.full((8, 128), i * TK + j, jnp.int32)
        nv, nc = [], []
        for u in range(UT):
            v = x_ref[pl.ds((j * UT + u) * 8, 8), :]
            a = jnp.abs(v)
            pred = a < bvs[u]
            nv.append(jnp.minimum(a, bvs[u]))
            nc.append(jnp.where(pred, codev, bcs[u]))
        return tuple(nv), tuple(nc)

    bvs, bcs = lax.fori_loop(
        0, TK, inner,
        (tuple(sv[u][...] for u in range(UT)),
         tuple(scd[u][...] for u in range(UT))))
    for u in range(UT):
        sv[u][...] = bvs[u]
        scd[u][...] = bcs[u]

    @pl.when(i == GRID - 1)
    def _():
        sub = lax.broadcasted_iota(jnp.int32, (8, 128), 0)
        col = lax.broadcasted_iota(jnp.int32, (8, 128), 1)
        best_v = bvs[0]
        best_p = S_SC + ((bcs[0] * UT + 0) * 8 + sub) * 128 + col
        for u in range(1, UT):
            p_u = S_SC + ((bcs[u] * UT + u) * 8 + sub) * 128 + col
            pred = (bvs[u] < best_v) | ((bvs[u] == best_v) & (p_u < best_p))
            best_v = jnp.where(pred, bvs[u], best_v)
            best_p = jnp.where(pred, p_u, best_p)
        vm = jnp.where((col & 1) == 0, best_v, jnp.inf)
        m = jnp.min(vm)
        sel = jnp.where(vm == m, best_p, I32_MAX)
        pbest = jnp.min(sel)
        val_ref[...] = jnp.reshape(m, (1, 1))
        idx_ref[...] = jnp.reshape(pbest, (1, 1))


def _merge_body(vals_ref, idxs_ref, tval_ref, tidx_ref, out_ref):
    v = vals_ref[...]
    p = idxs_ref[...]
    tv = tval_ref[0, 0]
    tp = tidx_ref[0, 0]
    col = lax.broadcasted_iota(jnp.int32, (NW, L), 1)
    even = (col & 1) == 0
    vm = jnp.where(even, v, jnp.inf)
    m = jnp.minimum(jnp.min(vm), tv)
    sel = jnp.where(vm == m, p, I32_MAX)
    p_sc = jnp.min(sel)
    p_tc = jnp.where(tv == m, tp, I32_MAX)
    p_best = jnp.minimum(p_sc, p_tc)
    out_ref[...] = jnp.reshape((p_best >> 1) + 1, (1, 1))


@jax.jit
def kernel(x):
    mesh = plsc.VectorSubcoreMesh(core_axis_name="c", subcore_axis_name="s")
    sc = functools.partial(
        pl.kernel,
        mesh=mesh,
        out_type=[
            jax.ShapeDtypeStruct((NW, L), jnp.float32),
            jax.ShapeDtypeStruct((NW, L), jnp.int32),
        ],
        scratch_types=[
            pltpu.VMEM((CHUNK,), jnp.float32),
            pltpu.VMEM((CHUNK,), jnp.float32),
            pltpu.VMEM((CHUNK,), jnp.float32),
            pltpu.VMEM((CHUNK,), jnp.float32),
            pltpu.VMEM((L,), jnp.float32),
            pltpu.VMEM((L,), jnp.int32),
            pltpu.SemaphoreType.DMA,
            pltpu.SemaphoreType.DMA,
            pltpu.SemaphoreType.DMA,
            pltpu.SemaphoreType.DMA,
        ],
    )(_sc_body)

    x2 = x.reshape(ROWS, 128)
    tval, tidx = pl.pallas_call(
        _tc_body,
        grid=(GRID,),
        in_specs=[pl.BlockSpec((BM, 128), lambda i: (ROW0 // BM + i, 0))],
        out_specs=[
            pl.BlockSpec((1, 1), lambda i: (0, 0)),
            pl.BlockSpec((1, 1), lambda i: (0, 0)),
        ],
        out_shape=[
            jax.ShapeDtypeStruct((1, 1), jnp.float32),
            jax.ShapeDtypeStruct((1, 1), jnp.int32),
        ],
        scratch_shapes=(
            [pltpu.VMEM((8, 128), jnp.float32) for _ in range(UT)]
            + [pltpu.VMEM((8, 128), jnp.int32) for _ in range(UT)]
        ),
    )(x2)
    vals, idxs = sc(x)

    out = pl.pallas_call(
        _merge_body,
        out_shape=jax.ShapeDtypeStruct((1, 1), jnp.int32),
    )(vals, idxs, tval, tidx)
    return out[0, 0]


# speedup vs baseline: 1.1679x; 1.0023x over previous
import functools

import jax
import jax.numpy as jnp
from jax import lax
from jax.experimental import pallas as pl
from jax.experimental.pallas import tpu as pltpu
from jax.experimental.pallas import tpu_sc as plsc

X_LEN = 33554432
I32_MAX = 2147483647

NC = 2
NS = 16
L = 16
NW = NC * NS
S_SC = 19922944
PER_W = S_SC // NW
CHUNK = 8192
NCHUNK = PER_W // CHUNK
NBUF = 4
U = 8
K = CHUNK // (L * U)

S_TC = X_LEN - S_SC
ROWS = X_LEN // 128
ROW0 = S_SC // 128
BM = 4096
GRID = S_TC // 128 // BM
UT = 8
TK = BM // (8 * UT)


def _sc_body(x_hbm, vals_hbm, idxs_hbm,
             buf0, buf1, buf2, buf3, vrow, irow,
             sem0, sem1, sem2, sem3):
    wid = lax.axis_index("s") * NC + lax.axis_index("c")
    base = wid * PER_W
    lane = lax.broadcasted_iota(jnp.int32, (L,), 0)
    bufs = (buf0, buf1, buf2, buf3)
    sems = (sem0, sem1, sem2, sem3)

    for b in range(NBUF):
        pltpu.async_copy(
            x_hbm.at[pl.ds(base + b * CHUNK, CHUNK)], bufs[b], sems[b])

    def make_inner(buf, ck):
        def inner(j, ic):
            bas, bcodes = ic
            codev = jnp.full((L,), ck + j, jnp.int32)
            nbas = []
            ncodes = []
            for u in range(U):
                v = buf[pl.ds(j * (L * U) + u * L, L)]
                a = jnp.abs(v)
                pred = a < bas[u]
                nbas.append(jnp.minimum(a, bas[u]))
                ncodes.append(jnp.where(pred, codev, bcodes[u]))
            return tuple(nbas), tuple(ncodes)
        return inner

    ngroup = NCHUNK // NBUF

    def group_step(g, carry):
        bas, bcodes = carry
        for b in range(NBUF):
            c = NBUF * g + b
            pltpu.make_async_copy(
                x_hbm.at[pl.ds(base, CHUNK)], bufs[b], sems[b]).wait()
            bas, bcodes = lax.fori_loop(
                0, K, make_inner(bufs[b], c * K), (bas, bcodes))

            @pl.when(g < ngroup - 1)
            def _():
                off_next = base + (c + NBUF) * CHUNK
                pltpu.async_copy(
                    x_hbm.at[pl.ds(off_next, CHUNK)], bufs[b], sems[b])
        return bas, bcodes

    init_a = tuple(jnp.full((L,), jnp.inf, jnp.float32) for _ in range(U))
    init_c = tuple(jnp.zeros((L,), jnp.int32) for _ in range(U))
    bas, bcodes = lax.fori_loop(0, ngroup, group_step, (init_a, init_c))

    best_a = bas[0]
    best_p = base + bcodes[0] * (L * U) + lane
    for u in range(1, U):
        p_u = base + bcodes[u] * (L * U) + (u * L) + lane
        pred = (bas[u] < best_a) | ((bas[u] == best_a) & (p_u < best_p))
        best_a = jnp.where(pred, bas[u], best_a)
        best_p = jnp.where(pred, p_u, best_p)

    vrow[...] = best_a
    irow[...] = best_p
    pltpu.sync_copy(vrow, vals_hbm.at[wid])
    pltpu.sync_copy(irow, idxs_hbm.at[wid])


def _tc_body(x_ref, val_ref, idx_ref, *state):
    i = pl.program_id(0)
    sv = state[:UT]
    scd = state[UT:]

    @pl.when(i == 0)
    def _():
        for u in range(UT):
            sv[u][...] = jnp.full((8, 128), jnp.inf, jnp.float32)
            scd[u][...] = jnp.zeros((8, 128), jnp.int32)

    def inner(j, carry):
        bvs, bcs = carry
        codev = jnp.full((8, 128), i * TK + j, jnp.int32)
        nv, nc = [], []
        for u in range(UT):
            v = x_ref[pl.ds((j * UT + u) * 8, 8), :]
            a = jnp.abs(v)
            pred = a < bvs[u]
            nv.append(jnp.minimum(a, bvs[u]))
            nc.append(jnp.where(pred, codev, bcs[u]))
        return tuple(nv), tuple(nc)

    bvs, bcs = lax.fori_loop(
        0, TK, inner,
        (tuple(sv[u][...] for u in range(UT)),
         tuple(scd[u][...] for u in range(UT))))
    for u in range(UT):
        sv[u][...] = bvs[u]
        scd[u][...] = bcs[u]

    @pl.when(i == GRID - 1)
    def _():
        sub = lax.broadcasted_iota(jnp.int32, (8, 128), 0)
        col = lax.broadcasted_iota(jnp.int32, (8, 128), 1)
        best_v = bvs[0]
        best_p = S_SC + ((bcs[0] * UT + 0) * 8 + sub) * 128 + col
        for u in range(1, UT):
            p_u = S_SC + ((bcs[u] * UT + u) * 8 + sub) * 128 + col
            pred = (bvs[u] < best_v) | ((bvs[u] == best_v) & (p_u < best_p))
            best_v = jnp.where(pred, bvs[u], best_v)
            best_p = jnp.where(pred, p_u, best_p)
        vm = jnp.where((col & 1) == 0, best_v, jnp.inf)
        m = jnp.min(vm)
        sel = jnp.where(vm == m, best_p, I32_MAX)
        pbest = jnp.min(sel)
        val_ref[...] = jnp.reshape(m, (1, 1))
        idx_ref[...] = jnp.reshape(pbest, (1, 1))


def _merge_body(vals_ref, idxs_ref, tval_ref, tidx_ref, out_ref):
    v = vals_ref[...]
    p = idxs_ref[...]
    tv = tval_ref[0, 0]
    tp = tidx_ref[0, 0]
    col = lax.broadcasted_iota(jnp.int32, (NW, L), 1)
    even = (col & 1) == 0
    vm = jnp.where(even, v, jnp.inf)
    m = jnp.minimum(jnp.min(vm), tv)
    sel = jnp.where(vm == m, p, I32_MAX)
    p_sc = jnp.min(sel)
    p_tc = jnp.where(tv == m, tp, I32_MAX)
    p_best = jnp.minimum(p_sc, p_tc)
    out_ref[...] = jnp.reshape((p_best >> 1) + 1, (1, 1))


@jax.jit
def kernel(x):
    mesh = plsc.VectorSubcoreMesh(core_axis_name="c", subcore_axis_name="s")
    sc = functools.partial(
        pl.kernel,
        mesh=mesh,
        out_type=[
            jax.ShapeDtypeStruct((NW, L), jnp.float32),
            jax.ShapeDtypeStruct((NW, L), jnp.int32),
        ],
        scratch_types=[
            pltpu.VMEM((CHUNK,), jnp.float32),
            pltpu.VMEM((CHUNK,), jnp.float32),
            pltpu.VMEM((CHUNK,), jnp.float32),
            pltpu.VMEM((CHUNK,), jnp.float32),
            pltpu.VMEM((L,), jnp.float32),
            pltpu.VMEM((L,), jnp.int32),
            pltpu.SemaphoreType.DMA,
            pltpu.SemaphoreType.DMA,
            pltpu.SemaphoreType.DMA,
            pltpu.SemaphoreType.DMA,
        ],
    )(_sc_body)

    x2 = x.reshape(ROWS, 128)
    tval, tidx = pl.pallas_call(
        _tc_body,
        grid=(GRID,),
        in_specs=[pl.BlockSpec((BM, 128), lambda i: (ROW0 // BM + i, 0))],
        out_specs=[
            pl.BlockSpec((1, 1), lambda i: (0, 0)),
            pl.BlockSpec((1, 1), lambda i: (0, 0)),
        ],
        out_shape=[
            jax.ShapeDtypeStruct((1, 1), jnp.float32),
            jax.ShapeDtypeStruct((1, 1), jnp.int32),
        ],
        scratch_shapes=(
            [pltpu.VMEM((8, 128), jnp.float32) for _ in range(UT)]
            + [pltpu.VMEM((8, 128), jnp.int32) for _ in range(UT)]
        ),
    )(x2)
    vals, idxs = sc(x)

    out = pl.pallas_call(
        _merge_body,
        out_shape=jax.ShapeDtypeStruct((1, 1), jnp.int32),
    )(vals, idxs, tval, tidx)
    return out[0, 0]
